# baseline (speedup 1.0000x reference)
import functools

import jax
import jax.numpy as jnp
from jax import lax
from jax.experimental import pallas as pl
from jax.experimental.pallas import tpu as pltpu
from jax.experimental.pallas import tpu_sc as plsc

_NC, _NS = 2, 16
_NW = _NC * _NS
_E_FLAT = 320000 * 16
_PER_W = _E_FLAT // _NW
_CHUNK = 16000
_NIT = _PER_W // _CHUNK

_mesh = plsc.VectorSubcoreMesh(core_axis_name="c", subcore_axis_name="s")


@functools.partial(
    pl.kernel,
    mesh=_mesh,
    out_type=jax.ShapeDtypeStruct((_E_FLAT,), jnp.float32),
    scratch_types=[pltpu.VMEM((_CHUNK,), jnp.float32)],
)
def _sc_copy(e_hbm, out_hbm, buf):
    wid = lax.axis_index("s") * _NC + lax.axis_index("c")
    base = wid * _PER_W
    for i in range(_NIT):
        off = base + i * _CHUNK
        pltpu.sync_copy(e_hbm.at[pl.ds(off, _CHUNK)], buf)
        pltpu.sync_copy(buf, out_hbm.at[pl.ds(off, _CHUNK)])


def kernel(x, edge_index, edge_attr):
    del edge_index
    n_edges, d_edge = edge_attr.shape
    e_flat = edge_attr.T.reshape(-1)
    e_out_flat = _sc_copy(e_flat)
    e_out = e_out_flat.reshape(d_edge, n_edges).T
    return (x, e_out)


# speedup vs baseline: 2.4610x; 2.4610x over previous
import functools

import jax
import jax.numpy as jnp
from jax import lax
from jax.experimental import pallas as pl
from jax.experimental.pallas import tpu as pltpu
from jax.experimental.pallas import tpu_sc as plsc

_NC, _NS = 2, 16
_NW = _NC * _NS
_X_FLAT = 10000 * 128
_PER_W = _X_FLAT // _NW
_C = 8000
_NIT = _PER_W // _C

_mesh = plsc.VectorSubcoreMesh(core_axis_name="c", subcore_axis_name="s")


@functools.partial(
    pl.kernel,
    mesh=_mesh,
    out_type=jax.ShapeDtypeStruct((_X_FLAT,), jnp.float32),
    scratch_types=[
        pltpu.VMEM((_C,), jnp.float32),
        pltpu.VMEM((_C,), jnp.float32),
        pltpu.SemaphoreType.DMA,
        pltpu.SemaphoreType.DMA,
        pltpu.SemaphoreType.DMA,
        pltpu.SemaphoreType.DMA,
    ],
)
def _sc_copy_x(x_hbm, out_hbm, buf0, buf1, si0, si1, so0, so1):
    wid = lax.axis_index("s") * _NC + lax.axis_index("c")
    base = wid * _PER_W
    bufs = (buf0, buf1)
    sin = (si0, si1)
    sout = (so0, so1)
    in_h = [None, None]
    out_h = [None, None]
    in_h[0] = pltpu.async_copy(x_hbm.at[pl.ds(base, _C)], bufs[0], sin[0])
    for i in range(_NIT):
        b = i % 2
        nxt = i + 1
        if nxt < _NIT:
            ob = nxt % 2
            if out_h[ob] is not None:
                out_h[ob].wait()
            in_h[ob] = pltpu.async_copy(
                x_hbm.at[pl.ds(base + nxt * _C, _C)], bufs[ob], sin[ob])
        in_h[b].wait()
        out_h[b] = pltpu.async_copy(
            bufs[b], out_hbm.at[pl.ds(base + i * _C, _C)], sout[b])
    for b in range(2):
        if out_h[b] is not None:
            out_h[b].wait()


def _copy_body(e_ref, e_out_ref):
    e_out_ref[...] = e_ref[...]


def kernel(x, edge_index, edge_attr):
    del edge_index
    n_nodes, d_feat = x.shape
    n_edges, d_edge = edge_attr.shape
    x_flat = x.reshape(-1)
    e_t = edge_attr.T

    x_out_flat = _sc_copy_x(x_flat)

    grid = 2
    be = n_edges // grid
    e_out_t = pl.pallas_call(
        _copy_body,
        grid=(grid,),
        out_shape=jax.ShapeDtypeStruct(e_t.shape, e_t.dtype),
        in_specs=[pl.BlockSpec((d_edge, be), lambda i: (0, i))],
        out_specs=pl.BlockSpec((d_edge, be), lambda i: (0, i)),
        compiler_params=pltpu.CompilerParams(
            dimension_semantics=("parallel",),
        ),
    )(e_t)
    return (x_out_flat.reshape(n_nodes, d_feat), e_out_t.T)


# speedup vs baseline: 4.9071x; 1.9940x over previous
import jax
import jax.numpy as jnp
from jax.experimental import pallas as pl
from jax.experimental.pallas import tpu as pltpu


def _copy_body(x_ref, e_ref, x_out_ref, e_out_ref):
    x_out_ref[...] = x_ref[...]
    e_out_ref[...] = e_ref[...]


def kernel(x, edge_index, edge_attr):
    del edge_index
    n_nodes, d_feat = x.shape
    n_edges, d_edge = edge_attr.shape
    e_t = edge_attr.T

    grid = 2
    bx = n_nodes // grid
    be = n_edges // grid

    x_out, e_out_t = pl.pallas_call(
        _copy_body,
        grid=(grid,),
        out_shape=(
            jax.ShapeDtypeStruct(x.shape, x.dtype),
            jax.ShapeDtypeStruct(e_t.shape, e_t.dtype),
        ),
        in_specs=[
            pl.BlockSpec((bx, d_feat), lambda i: (i, 0)),
            pl.BlockSpec((d_edge, be), lambda i: (0, i)),
        ],
        out_specs=(
            pl.BlockSpec((bx, d_feat), lambda i: (i, 0)),
            pl.BlockSpec((d_edge, be), lambda i: (0, i)),
        ),
        compiler_params=pltpu.CompilerParams(
            dimension_semantics=("parallel",),
        ),
    )(x, e_t)
    return (x_out, e_out_t.T)
